# Initial kernel scaffold; baseline (speedup 1.0000x reference)
import functools

import jax
import jax.numpy as jnp
from jax import lax
from jax.experimental import pallas as pl
from jax.experimental.pallas import tpu as pltpu
from jax.experimental.pallas import tpu_sc as plsc

B, L1, L2, D = 8, 512, 2048, 3
C = L2 * D
NC, NS = 2, 16
NW = NC * NS
WPB = NW // B
CPW = C // WPB
R = 32
NCHUNK = L1 // R
LANES = 16
NGROUP = CPW // LANES
G = 4
NJJ = NGROUP // G


def _ins3(m1, m2, m3, x):
    t1 = jnp.minimum(m1, x)
    r1 = jnp.maximum(m1, x)
    t2 = jnp.minimum(m2, r1)
    r2 = jnp.maximum(m2, r1)
    t3 = jnp.minimum(m3, r2)
    return t1, t2, t3


def _body(x_hbm, wb_hbm, out_hbm, buf0, buf1, m1, m2, m3, outv, wv,
          sem0, sem1, semw):
    cid = lax.axis_index("c")
    sid = lax.axis_index("s")
    wid = sid * NC + cid
    b_idx = wid // WPB
    c0 = (wid % WPB) * CPW

    pltpu.async_copy(wb_hbm, wv, semw).wait()

    inf = jnp.full((LANES,), jnp.inf, jnp.float32)

    def init_j(j, carry):
        m1[pl.ds(j * LANES, LANES)] = inf
        m2[pl.ds(j * LANES, LANES)] = inf
        m3[pl.ds(j * LANES, LANES)] = inf
        return carry

    lax.fori_loop(0, NGROUP, init_j, 0)

    bufs = (buf0, buf1)
    sems = (sem0, sem1)

    def start(g):
        return pltpu.async_copy(
            x_hbm.at[b_idx, pl.ds(g * R, R), pl.ds(c0, CPW)],
            bufs[g % 2], sems[g % 2])

    pending = start(0)
    for g in range(NCHUNK):
        pending.wait()
        nxt = start(g + 1) if g + 1 < NCHUNK else None
        buf = bufs[g % 2]

        def jj_body(jj, carry, buf=buf):
            base = jj * (G * LANES)
            st = []
            for gi in range(G):
                off = base + gi * LANES
                st += [m1[pl.ds(off, LANES)],
                       m2[pl.ds(off, LANES)],
                       m3[pl.ds(off, LANES)]]

            def r_body(r, st):
                st = list(st)
                for gi in range(G):
                    off = base + gi * LANES
                    x = buf[r, pl.ds(off, LANES)]
                    a, b2, c2 = _ins3(st[3 * gi], st[3 * gi + 1],
                                      st[3 * gi + 2], x)
                    st[3 * gi], st[3 * gi + 1], st[3 * gi + 2] = a, b2, c2
                return tuple(st)

            st = lax.fori_loop(0, R, r_body, tuple(st))
            for gi in range(G):
                off = base + gi * LANES
                m1[pl.ds(off, LANES)] = st[3 * gi]
                m2[pl.ds(off, LANES)] = st[3 * gi + 1]
                m3[pl.ds(off, LANES)] = st[3 * gi + 2]
            return carry

        lax.fori_loop(0, NJJ, jj_body, 0)
        pending = nxt

    iota = lax.iota(jnp.int32, LANES)
    ms = (m1, m2, m3)

    def blk_body(blk, carry):
        lbase = blk * LANES
        for co in range(D):
            acc = jnp.zeros((LANES,), jnp.float32) + wv[27 + co]
            for k in range(3):
                for dd in range(D):
                    idx = (lbase + iota) * D + dd
                    gv = plsc.load_gather(ms[k], [idx])
                    acc = acc + gv * wv[(k * D + dd) * D + co]
            oidx = (lbase + iota) * D + co
            plsc.store_scatter(outv, [oidx], acc)
        return carry

    lax.fori_loop(0, CPW // (D * LANES), blk_body, 0)
    pltpu.sync_copy(outv, out_hbm.at[b_idx, pl.ds(c0, CPW)])


@functools.partial(jax.jit)
def kernel(local_decisions, W, b):
    x = local_decisions.reshape(B, L1, C)
    wb = jnp.concatenate(
        [W.reshape(-1), b, jnp.zeros((2,), jnp.float32)]).astype(jnp.float32)
    mesh = plsc.VectorSubcoreMesh(
        core_axis_name="c", subcore_axis_name="s",
        num_cores=NC, num_subcores=NS)
    out = pl.kernel(
        _body,
        out_type=jax.ShapeDtypeStruct((B, C), jnp.float32),
        mesh=mesh,
        scratch_types=[
            pltpu.VMEM((R, CPW), jnp.float32),
            pltpu.VMEM((R, CPW), jnp.float32),
            pltpu.VMEM((CPW,), jnp.float32),
            pltpu.VMEM((CPW,), jnp.float32),
            pltpu.VMEM((CPW,), jnp.float32),
            pltpu.VMEM((CPW,), jnp.float32),
            pltpu.VMEM((32,), jnp.float32),
            pltpu.SemaphoreType.DMA,
            pltpu.SemaphoreType.DMA,
            pltpu.SemaphoreType.DMA,
        ],
    )(x, wb)
    return out.reshape(B, L2, D)


# speedup vs baseline: 14.8527x; 14.8527x over previous
import functools

import jax
import jax.numpy as jnp
from jax import lax
from jax.experimental import pallas as pl
from jax.experimental.pallas import tpu as pltpu
from jax.experimental.pallas import tpu_sc as plsc

B, L1, L2, D = 8, 512, 2048, 3
C = L2 * D
NC, NS = 2, 16
NW = NC * NS
WPB = NW // B
CPW = C // WPB
R = 32
NCHUNK = L1 // R
LANES = 16
NGROUP = CPW // LANES
G = 4
NJJ = NGROUP // G


def _ins3(m1, m2, m3, x):
    t1 = jnp.minimum(m1, x)
    r1 = jnp.maximum(m1, x)
    t2 = jnp.minimum(m2, r1)
    r2 = jnp.maximum(m2, r1)
    t3 = jnp.minimum(m3, r2)
    return t1, t2, t3


def _body(x_hbm, wb_hbm, out_hbm, buf0, buf1, m1, m2, m3, outv, wv,
          sem0, sem1, semw):
    cid = lax.axis_index("c")
    sid = lax.axis_index("s")
    wid = sid * NC + cid
    b_idx = wid // WPB
    c0 = (wid % WPB) * CPW

    pltpu.async_copy(wb_hbm, wv, semw).wait()

    inf = jnp.full((LANES,), jnp.inf, jnp.float32)

    def init_j(j, carry):
        m1[pl.ds(j * LANES, LANES)] = inf
        m2[pl.ds(j * LANES, LANES)] = inf
        m3[pl.ds(j * LANES, LANES)] = inf
        return carry

    lax.fori_loop(0, NGROUP, init_j, 0)

    bufs = (buf0, buf1)
    sems = (sem0, sem1)

    def start(g):
        return pltpu.async_copy(
            x_hbm.at[b_idx, pl.ds(g * R, R), pl.ds(c0, CPW)],
            bufs[g % 2], sems[g % 2])

    pending = start(0)
    for g in range(NCHUNK):
        pending.wait()
        nxt = start(g + 1) if g + 1 < NCHUNK else None
        buf = bufs[g % 2]

        def jj_body(jj, carry, buf=buf):
            base = jj * (G * LANES)
            st = []
            for gi in range(G):
                off = base + gi * LANES
                st += [m1[pl.ds(off, LANES)],
                       m2[pl.ds(off, LANES)],
                       m3[pl.ds(off, LANES)]]

            def r_body(r, st):
                st = list(st)
                for gi in range(G):
                    off = base + gi * LANES
                    x = buf[r, pl.ds(off, LANES)]
                    a, b2, c2 = _ins3(st[3 * gi], st[3 * gi + 1],
                                      st[3 * gi + 2], x)
                    st[3 * gi], st[3 * gi + 1], st[3 * gi + 2] = a, b2, c2
                return tuple(st)

            st = lax.fori_loop(0, R, r_body, tuple(st))
            for gi in range(G):
                off = base + gi * LANES
                m1[pl.ds(off, LANES)] = st[3 * gi]
                m2[pl.ds(off, LANES)] = st[3 * gi + 1]
                m3[pl.ds(off, LANES)] = st[3 * gi + 2]
            return carry

        lax.fori_loop(0, NJJ, jj_body, 0)
        pending = nxt

    iota = lax.iota(jnp.int32, LANES)
    ms = (m1, m2, m3)
    w_lo = wv[pl.ds(0, LANES)]
    w_hi = wv[pl.ds(LANES, LANES)]

    def _w(i):
        return w_lo[i] if i < LANES else w_hi[i - LANES]

    def blk_body(blk, carry):
        lbase = blk * LANES
        for co in range(D):
            acc = jnp.zeros((LANES,), jnp.float32) + _w(27 + co)
            for k in range(3):
                for dd in range(D):
                    idx = (lbase + iota) * D + dd
                    gv = plsc.load_gather(ms[k], [idx])
                    acc = acc + gv * _w((k * D + dd) * D + co)
            oidx = (lbase + iota) * D + co
            plsc.store_scatter(outv, [oidx], acc)
        return carry

    lax.fori_loop(0, CPW // (D * LANES), blk_body, 0)
    pltpu.sync_copy(outv, out_hbm.at[b_idx, pl.ds(c0, CPW)])


@functools.partial(jax.jit)
def kernel(local_decisions, W, b):
    x = local_decisions.reshape(B, L1, C)
    wb = jnp.concatenate(
        [W.reshape(-1), b, jnp.zeros((2,), jnp.float32)]).astype(jnp.float32)
    mesh = plsc.VectorSubcoreMesh(
        core_axis_name="c", subcore_axis_name="s",
        num_cores=NC, num_subcores=NS)
    out = pl.kernel(
        _body,
        out_type=jax.ShapeDtypeStruct((B, C), jnp.float32),
        mesh=mesh,
        compiler_params=pltpu.CompilerParams(needs_layout_passes=False),
        scratch_types=[
            pltpu.VMEM((R, CPW), jnp.float32),
            pltpu.VMEM((R, CPW), jnp.float32),
            pltpu.VMEM((CPW,), jnp.float32),
            pltpu.VMEM((CPW,), jnp.float32),
            pltpu.VMEM((CPW,), jnp.float32),
            pltpu.VMEM((CPW,), jnp.float32),
            pltpu.VMEM((32,), jnp.float32),
            pltpu.SemaphoreType.DMA,
            pltpu.SemaphoreType.DMA,
            pltpu.SemaphoreType.DMA,
        ],
    )(x, wb)
    return out.reshape(B, L2, D)


# speedup vs baseline: 14.9134x; 1.0041x over previous
import functools

import jax
import jax.numpy as jnp
from jax import lax
from jax.experimental import pallas as pl
from jax.experimental.pallas import tpu as pltpu
from jax.experimental.pallas import tpu_sc as plsc

B, L1, L2, D = 8, 512, 2048, 3
C = L2 * D
NC, NS = 2, 16
NW = NC * NS
WPB = NW // B
CPW = C // WPB
R = 32
NCHUNK = L1 // R
LANES = 16
NGROUP = CPW // LANES
G = 4
NJJ = NGROUP // G


def _ins3(m1, m2, m3, x):
    t1 = jnp.minimum(m1, x)
    r1 = jnp.maximum(m1, x)
    t2 = jnp.minimum(m2, r1)
    r2 = jnp.maximum(m2, r1)
    t3 = jnp.minimum(m3, r2)
    return t1, t2, t3


def _body(x_hbm, wb_hbm, out_hbm, buf0, buf1, m1, m2, m3, outv, wv,
          sem0, sem1, semw):
    cid = lax.axis_index("c")
    sid = lax.axis_index("s")
    wid = sid * NC + cid
    b_idx = wid // WPB
    c0 = (wid % WPB) * CPW

    pltpu.async_copy(wb_hbm, wv, semw).wait()

    inf = jnp.full((LANES,), jnp.inf, jnp.float32)

    def init_j(j, carry):
        m1[pl.ds(j * LANES, LANES)] = inf
        m2[pl.ds(j * LANES, LANES)] = inf
        m3[pl.ds(j * LANES, LANES)] = inf
        return carry

    lax.fori_loop(0, NGROUP, init_j, 0)

    bufs = (buf0, buf1)
    sems = (sem0, sem1)

    def start(g):
        return pltpu.async_copy(
            x_hbm.at[b_idx, pl.ds(g * R, R), pl.ds(c0, CPW)],
            bufs[g % 2], sems[g % 2])

    pending = start(0)
    for g in range(NCHUNK):
        pending.wait()
        nxt = start(g + 1) if g + 1 < NCHUNK else None
        buf = bufs[g % 2]

        def jj_body(jj, carry, buf=buf):
            base = jj * (G * LANES)
            st = []
            for gi in range(G):
                off = base + gi * LANES
                st += [m1[pl.ds(off, LANES)],
                       m2[pl.ds(off, LANES)],
                       m3[pl.ds(off, LANES)]]

            def r_body(r, st):
                st = list(st)
                for gi in range(G):
                    off = base + gi * LANES
                    x = buf[r, pl.ds(off, LANES)]
                    a, b2, c2 = _ins3(st[3 * gi], st[3 * gi + 1],
                                      st[3 * gi + 2], x)
                    st[3 * gi], st[3 * gi + 1], st[3 * gi + 2] = a, b2, c2
                return tuple(st)

            st = lax.fori_loop(0, R, r_body, tuple(st), unroll=4)
            for gi in range(G):
                off = base + gi * LANES
                m1[pl.ds(off, LANES)] = st[3 * gi]
                m2[pl.ds(off, LANES)] = st[3 * gi + 1]
                m3[pl.ds(off, LANES)] = st[3 * gi + 2]
            return carry

        lax.fori_loop(0, NJJ, jj_body, 0)
        pending = nxt

    iota = lax.iota(jnp.int32, LANES)
    ms = (m1, m2, m3)
    w_lo = wv[pl.ds(0, LANES)]
    w_hi = wv[pl.ds(LANES, LANES)]

    def _w(i):
        return w_lo[i] if i < LANES else w_hi[i - LANES]

    def blk_body(blk, carry):
        lbase = blk * LANES
        for co in range(D):
            acc = jnp.zeros((LANES,), jnp.float32) + _w(27 + co)
            for k in range(3):
                for dd in range(D):
                    idx = (lbase + iota) * D + dd
                    gv = plsc.load_gather(ms[k], [idx])
                    acc = acc + gv * _w((k * D + dd) * D + co)
            oidx = (lbase + iota) * D + co
            plsc.store_scatter(outv, [oidx], acc)
        return carry

    lax.fori_loop(0, CPW // (D * LANES), blk_body, 0)
    pltpu.sync_copy(outv, out_hbm.at[b_idx, pl.ds(c0, CPW)])


@functools.partial(jax.jit)
def kernel(local_decisions, W, b):
    x = local_decisions.reshape(B, L1, C)
    wb = jnp.concatenate(
        [W.reshape(-1), b, jnp.zeros((2,), jnp.float32)]).astype(jnp.float32)
    mesh = plsc.VectorSubcoreMesh(
        core_axis_name="c", subcore_axis_name="s",
        num_cores=NC, num_subcores=NS)
    out = pl.kernel(
        _body,
        out_type=jax.ShapeDtypeStruct((B, C), jnp.float32),
        mesh=mesh,
        compiler_params=pltpu.CompilerParams(needs_layout_passes=False),
        scratch_types=[
            pltpu.VMEM((R, CPW), jnp.float32),
            pltpu.VMEM((R, CPW), jnp.float32),
            pltpu.VMEM((CPW,), jnp.float32),
            pltpu.VMEM((CPW,), jnp.float32),
            pltpu.VMEM((CPW,), jnp.float32),
            pltpu.VMEM((CPW,), jnp.float32),
            pltpu.VMEM((32,), jnp.float32),
            pltpu.SemaphoreType.DMA,
            pltpu.SemaphoreType.DMA,
            pltpu.SemaphoreType.DMA,
        ],
    )(x, wb)
    return out.reshape(B, L2, D)
